# Initial kernel scaffold; baseline (speedup 1.0000x reference)
import functools

import jax
import jax.numpy as jnp
from jax import lax
from jax.experimental import pallas as pl
from jax.experimental.pallas import tpu as pltpu
from jax.experimental.pallas import tpu_sc as plsc

F = 26
OUT = 16
B = 16384
NW = 32
BPT = B // NW
C = 128
NCHUNK = BPT // C


def _field_linear_sc(xt, weight, off2, bias):
    mesh = plsc.VectorSubcoreMesh(core_axis_name="c", subcore_axis_name="s")

    @functools.partial(
        pl.kernel,
        out_type=jax.ShapeDtypeStruct((B, OUT), jnp.float32),
        mesh=mesh,
        scratch_types=[
            pltpu.VMEM((F, OUT), jnp.int32),
            pltpu.VMEM((OUT,), jnp.float32),
            pltpu.VMEM((F, C), jnp.int32),
            pltpu.VMEM((F, C), jnp.int32),
            pltpu.VMEM((F, C, OUT), jnp.float32),
            pltpu.VMEM((C, OUT), jnp.float32),
            pltpu.SemaphoreType.DMA,
        ],
    )
    def k(xt_hbm, w_hbm, off_hbm, bias_hbm, out_hbm,
          off_v, bias_v, xv, idx_v, gbuf, outb, sem):
        cid = lax.axis_index("c")
        sid = lax.axis_index("s")
        wid = sid * 2 + cid
        pltpu.sync_copy(off_hbm, off_v)
        pltpu.sync_copy(bias_hbm, bias_v)
        bias_vec = bias_v[:]

        def chunk_body(ci, carry):
            base = wid * BPT + ci * C
            pltpu.sync_copy(xt_hbm.at[:, pl.ds(base, C)], xv)
            for f in range(F):
                offb = off_v[f, :]
                for j in range(C // OUT):
                    idx_v[f, pl.ds(j * OUT, OUT)] = (
                        xv[f, pl.ds(j * OUT, OUT)] + offb)
            descs = [
                pltpu.async_copy(w_hbm.at[idx_v.at[f]], gbuf.at[f], sem)
                for f in range(F)
            ]
            for dsc in descs:
                dsc.wait()

            def row_body(j, c2):
                acc = bias_vec
                for f in range(F):
                    acc = acc + gbuf[f, j, :]
                outb[j, :] = acc
                return c2

            lax.fori_loop(0, C, row_body, 0)
            pltpu.sync_copy(outb, out_hbm.at[pl.ds(base, C), :])
            return carry

        lax.fori_loop(0, NCHUNK, chunk_body, 0)

    return k(xt, weight, off2, bias)


def kernel(x, weight, bias, offset):
    xt = jnp.ascontiguousarray(x.T)
    off2 = jnp.broadcast_to(offset[:, None], (F, OUT))
    off2 = jnp.ascontiguousarray(off2.astype(jnp.int32))
    return _field_linear_sc(xt, weight, off2, bias.astype(jnp.float32))


# speedup vs baseline: 1.1518x; 1.1518x over previous
import functools

import jax
import jax.numpy as jnp
from jax import lax
from jax.experimental import pallas as pl
from jax.experimental.pallas import tpu as pltpu
from jax.experimental.pallas import tpu_sc as plsc

F = 26
OUT = 16
B = 16384
NW = 32
BPT = B // NW
C = 128
NCHUNK = BPT // C


def _field_linear_sc(xt, weight, off2, bias):
    mesh = plsc.VectorSubcoreMesh(core_axis_name="c", subcore_axis_name="s")

    @functools.partial(
        pl.kernel,
        out_type=jax.ShapeDtypeStruct((B, OUT), jnp.float32),
        mesh=mesh,
        compiler_params=pltpu.CompilerParams(use_tc_tiling_on_sc=False),
        scratch_types=[
            pltpu.VMEM((F, OUT), jnp.int32),
            pltpu.VMEM((OUT,), jnp.float32),
            pltpu.VMEM((F, C), jnp.int32),
            pltpu.VMEM((F, C), jnp.int32),
            pltpu.VMEM((F, C, OUT), jnp.float32),
            pltpu.VMEM((C, OUT), jnp.float32),
            pltpu.SemaphoreType.DMA,
        ],
    )
    def k(xt_hbm, w_hbm, off_hbm, bias_hbm, out_hbm,
          off_v, bias_v, xv, idx_v, gbuf, outb, sem):
        cid = lax.axis_index("c")
        sid = lax.axis_index("s")
        wid = sid * 2 + cid
        pltpu.sync_copy(off_hbm, off_v)
        pltpu.sync_copy(bias_hbm, bias_v)
        bias_vec = bias_v[:]

        def chunk_body(ci, carry):
            base = wid * BPT + ci * C
            pltpu.sync_copy(xt_hbm.at[:, pl.ds(base, C)], xv)
            for f in range(F):
                offb = off_v[f, :]
                for j in range(C // OUT):
                    idx_v[f, pl.ds(j * OUT, OUT)] = (
                        xv[f, pl.ds(j * OUT, OUT)] + offb)
            descs = [
                pltpu.async_copy(w_hbm.at[idx_v.at[f]], gbuf.at[f], sem)
                for f in range(F)
            ]
            for dsc in descs:
                dsc.wait()

            def row_body(j, c2):
                acc = bias_vec
                for f in range(F):
                    acc = acc + gbuf[f, j, :]
                outb[j, :] = acc
                return c2

            lax.fori_loop(0, C, row_body, 0)
            pltpu.sync_copy(outb, out_hbm.at[pl.ds(base, C), :])
            return carry

        lax.fori_loop(0, NCHUNK, chunk_body, 0)

    return k(xt, weight, off2, bias)


def kernel(x, weight, bias, offset):
    xt = x.T.copy()
    off2 = jnp.broadcast_to(offset[:, None], (F, OUT)).astype(jnp.int32).copy()
    return _field_linear_sc(xt, weight, off2, bias.astype(jnp.float32))
